# baseline (speedup 1.0000x reference)
import jax
import jax.numpy as jnp
from jax.experimental import pallas as pl
from jax.experimental.pallas import tpu as pltpu

S, D, E, F, FS = 2048, 2048, 8, 1024, 2048
T = 256
MAXB = (S * 2) // T + E - 1
P = MAXB * T
POUT = (MAXB + 1) * T

BF = jnp.bfloat16
F32 = jnp.float32


def _dot_t(a, b):
    return jax.lax.dot_general(a, b, (((1,), (1,)), ((), ())),
                               preferred_element_type=F32)


def _router_body(x_ref, wg_ref, e1_ref, e2_ref, w1_ref, w2_ref):
    logits = _dot_t(x_ref[...], wg_ref[...].astype(BF))
    m = jnp.max(logits, axis=-1, keepdims=True)
    ex = jnp.exp(logits - m)
    g = ex / jnp.sum(ex, axis=-1, keepdims=True)
    iota8 = jax.lax.broadcasted_iota(jnp.int32, g.shape, 1)
    g1 = jnp.max(g, axis=-1)
    i1 = jnp.min(jnp.where(g == g1[:, None], iota8, E), axis=-1)
    gm = jnp.where(iota8 == i1[:, None], -jnp.inf, g)
    g2 = jnp.max(gm, axis=-1)
    i2 = jnp.min(jnp.where(gm == g2[:, None], iota8, E), axis=-1)
    s = g1 + g2
    e1_ref[...] = i1
    e2_ref[...] = i2
    w1_ref[...] = g1 / s
    w2_ref[...] = g2 / s


def _router(xb, wg):
    bt = 256
    return pl.pallas_call(
        _router_body,
        grid=(S // bt,),
        in_specs=[
            pl.BlockSpec((bt, D), lambda t: (t, 0)),
            pl.BlockSpec((E, D), lambda t: (0, 0)),
        ],
        out_specs=[
            pl.BlockSpec((bt,), lambda t: (t,)),
            pl.BlockSpec((bt,), lambda t: (t,)),
            pl.BlockSpec((bt,), lambda t: (t,)),
            pl.BlockSpec((bt,), lambda t: (t,)),
        ],
        out_shape=[
            jax.ShapeDtypeStruct((S,), jnp.int32),
            jax.ShapeDtypeStruct((S,), jnp.int32),
            jax.ShapeDtypeStruct((S,), F32),
            jax.ShapeDtypeStruct((S,), F32),
        ],
    )(xb, wg)


def _s1_body(x_ref, gw_ref, uw_ref, h_ref, gws_ref, uws_ref):
    @pl.when(pl.program_id(1) == 0)
    def _():
        gws_ref[...] = gw_ref[...].astype(BF)
        uws_ref[...] = uw_ref[...].astype(BF)

    a = _dot_t(x_ref[...], gws_ref[...])
    b = _dot_t(x_ref[...], uws_ref[...])
    h_ref[...] = (a * jax.nn.sigmoid(a) * b).astype(BF)


def _shared_h(xb, gw, uw):
    bt, bf = 256, 1024
    return pl.pallas_call(
        _s1_body,
        grid=(FS // bf, S // bt),
        in_specs=[
            pl.BlockSpec((bt, D), lambda f, t: (t, 0)),
            pl.BlockSpec((bf, D), lambda f, t: (f, 0)),
            pl.BlockSpec((bf, D), lambda f, t: (f, 0)),
        ],
        out_specs=pl.BlockSpec((bt, bf), lambda f, t: (t, f)),
        out_shape=jax.ShapeDtypeStruct((S, FS), BF),
        scratch_shapes=[pltpu.VMEM((bf, D), BF), pltpu.VMEM((bf, D), BF)],
    )(xb, gw, uw)


def _s2_body(h_ref, dw_ref, o_ref, dws_ref):
    @pl.when(pl.program_id(0) == 0)
    def _():
        dws_ref[...] = dw_ref[...].astype(BF)

    o_ref[...] = _dot_t(h_ref[...], dws_ref[...])


def _shared_out(h, dw):
    bt = 256
    return pl.pallas_call(
        _s2_body,
        grid=(S // bt,),
        in_specs=[
            pl.BlockSpec((bt, FS), lambda t: (t, 0)),
            pl.BlockSpec((D, FS), lambda t: (0, 0)),
        ],
        out_specs=pl.BlockSpec((bt, D), lambda t: (t, 0)),
        out_shape=jax.ShapeDtypeStruct((S, D), F32),
        scratch_shapes=[pltpu.VMEM((D, FS), BF)],
    )(h, dw)


def _g1_body(te_ref, ob_ref, tv_ref, xg_ref, w1_ref, w3_ref, h_ref,
             w1s_ref, w3s_ref):
    t = pl.program_id(0)
    tm = jnp.maximum(t - 1, 0)
    new_w = (t == 0) | (te_ref[t] != te_ref[tm])

    @pl.when(new_w)
    def _():
        w1s_ref[...] = w1_ref[0].astype(BF)
        w3s_ref[...] = w3_ref[0].astype(BF)

    @pl.when(tv_ref[t] == 1)
    def _():
        xb = xg_ref[...].astype(BF)
        a = _dot_t(xb, w1s_ref[...])
        b = _dot_t(xb, w3s_ref[...])
        h_ref[...] = (a * jax.nn.sigmoid(a) * b).astype(BF)


def _grouped_h(xg, w1, w3, te, ob, tv):
    spec = pltpu.PrefetchScalarGridSpec(
        num_scalar_prefetch=3,
        grid=(MAXB,),
        in_specs=[
            pl.BlockSpec((T, D), lambda t, te, ob, tv: (t, 0)),
            pl.BlockSpec((1, F, D), lambda t, te, ob, tv: (te[t], 0, 0)),
            pl.BlockSpec((1, F, D), lambda t, te, ob, tv: (te[t], 0, 0)),
        ],
        out_specs=pl.BlockSpec((T, F), lambda t, te, ob, tv: (t, 0)),
        scratch_shapes=[pltpu.VMEM((F, D), BF), pltpu.VMEM((F, D), BF)],
    )
    return pl.pallas_call(
        _g1_body,
        grid_spec=spec,
        out_shape=jax.ShapeDtypeStruct((P, F), BF),
    )(te, ob, tv, xg, w1, w3)


def _g2_body(te_ref, ob_ref, tv_ref, h_ref, w2_ref, ws_ref, y_ref, w2s_ref):
    t = pl.program_id(0)
    tm = jnp.maximum(t - 1, 0)
    new_w = (t == 0) | (te_ref[t] != te_ref[tm])

    @pl.when(new_w)
    def _():
        w2s_ref[...] = w2_ref[0].astype(BF)

    @pl.when(tv_ref[t] == 1)
    def _():
        y = _dot_t(h_ref[...], w2s_ref[...])
        y_ref[...] = y * ws_ref[0, 0][:, None]


def _grouped_out(h, w2, ws3d, te, ob, tv):
    spec = pltpu.PrefetchScalarGridSpec(
        num_scalar_prefetch=3,
        grid=(MAXB,),
        in_specs=[
            pl.BlockSpec((T, F), lambda t, te, ob, tv: (t, 0)),
            pl.BlockSpec((1, D, F), lambda t, te, ob, tv: (te[t], 0, 0)),
            pl.BlockSpec((1, 1, T), lambda t, te, ob, tv: (t, 0, 0)),
        ],
        out_specs=pl.BlockSpec((T, D), lambda t, te, ob, tv: (ob[t], 0)),
        scratch_shapes=[pltpu.VMEM((D, F), BF)],
    )
    return pl.pallas_call(
        _g2_body,
        grid_spec=spec,
        out_shape=jax.ShapeDtypeStruct((POUT, D), F32),
    )(te, ob, tv, h, w2, ws3d)


def _dispatch_plan(e1, e2, wA, wB):
    ef = jnp.stack([e1, e2], axis=1).reshape(-1)
    wf = jnp.stack([wA, wB], axis=1).reshape(-1)
    oh = (ef[:, None] == jnp.arange(E)[None, :]).astype(jnp.int32)
    ranks = jnp.cumsum(oh, axis=0)
    rank = jnp.sum(ranks * oh, axis=1) - 1
    counts = ranks[-1]
    nblk = (counts + T - 1) // T
    ends = jnp.cumsum(nblk)
    startblk = ends - nblk
    ppos = startblk[ef] * T + rank
    tok = jnp.arange(2 * S, dtype=jnp.int32) // 2
    disp = jnp.zeros((P,), jnp.int32).at[ppos].set(
        tok, mode="drop", unique_indices=True)
    ws = jnp.zeros((P,), F32).at[ppos].set(
        wf, mode="drop", unique_indices=True)
    p0 = ppos[0::2]
    p1 = ppos[1::2]
    tvec = jnp.arange(MAXB, dtype=jnp.int32)
    used = ends[-1]
    te = jnp.minimum(jnp.searchsorted(ends, tvec, side="right"),
                     E - 1).astype(jnp.int32)
    tv = (tvec < used).astype(jnp.int32)
    ob = jnp.where(tv == 1, tvec, MAXB).astype(jnp.int32)
    return disp, ws, p0, p1, te, tv, ob


def kernel(hidden_states, wg, shared_gate_w, shared_up_w, shared_down_w, w1, w2, w3):
    B = hidden_states.shape[0]
    x = hidden_states.reshape(S, D)
    xb = x.astype(BF)

    e1, e2, wA, wB = _router(xb, wg)
    disp, ws, p0, p1, te, tv, ob = _dispatch_plan(e1, e2, wA, wB)

    hs = _shared_h(xb, shared_gate_w, shared_up_w)
    shared = _shared_out(hs, shared_down_w)

    xg = jnp.take(x, disp, axis=0)
    h = _grouped_h(xg, w1, w3, te, ob, tv)
    y = _grouped_out(h, w2, ws.reshape(MAXB, 1, T), te, ob, tv)

    routed = jnp.take(y, p0, axis=0) + jnp.take(y, p1, axis=0)
    return (shared + routed).reshape(B, S, D)


# speedup vs baseline: 1.0188x; 1.0188x over previous
import jax
import jax.numpy as jnp
from jax.experimental import pallas as pl
from jax.experimental.pallas import tpu as pltpu

S, D, E, F, FS = 2048, 2048, 8, 1024, 2048
T = 256
MAXB = (S * 2) // T + E - 1
P = MAXB * T
POUT = (MAXB + 1) * T

BF = jnp.bfloat16
F32 = jnp.float32


def _dot_t(a, b):
    return jax.lax.dot_general(a, b, (((1,), (1,)), ((), ())),
                               preferred_element_type=F32)


def _router_body(x_ref, wg_ref, e1_ref, e2_ref, w1_ref, w2_ref):
    logits = _dot_t(x_ref[...], wg_ref[...].astype(BF))
    m = jnp.max(logits, axis=-1, keepdims=True)
    ex = jnp.exp(logits - m)
    g = ex / jnp.sum(ex, axis=-1, keepdims=True)
    iota8 = jax.lax.broadcasted_iota(jnp.int32, g.shape, 1)
    g1 = jnp.max(g, axis=-1)
    i1 = jnp.min(jnp.where(g == g1[:, None], iota8, E), axis=-1)
    gm = jnp.where(iota8 == i1[:, None], -jnp.inf, g)
    g2 = jnp.max(gm, axis=-1)
    i2 = jnp.min(jnp.where(gm == g2[:, None], iota8, E), axis=-1)
    s = g1 + g2
    e1_ref[...] = i1
    e2_ref[...] = i2
    w1_ref[...] = g1 / s
    w2_ref[...] = g2 / s


def _router(xb, wg):
    bt = 256
    return pl.pallas_call(
        _router_body,
        grid=(S // bt,),
        in_specs=[
            pl.BlockSpec((bt, D), lambda t: (t, 0)),
            pl.BlockSpec((E, D), lambda t: (0, 0)),
        ],
        out_specs=[
            pl.BlockSpec((bt,), lambda t: (t,)),
            pl.BlockSpec((bt,), lambda t: (t,)),
            pl.BlockSpec((bt,), lambda t: (t,)),
            pl.BlockSpec((bt,), lambda t: (t,)),
        ],
        out_shape=[
            jax.ShapeDtypeStruct((S,), jnp.int32),
            jax.ShapeDtypeStruct((S,), jnp.int32),
            jax.ShapeDtypeStruct((S,), F32),
            jax.ShapeDtypeStruct((S,), F32),
        ],
    )(xb, wg)


def _s1_body(x_ref, gw_ref, uw_ref, h_ref):
    a = _dot_t(x_ref[...], gw_ref[...].astype(BF))
    b = _dot_t(x_ref[...], uw_ref[...].astype(BF))
    h_ref[...] = (a * jax.nn.sigmoid(a) * b).astype(BF)


def _shared_h(xb, gw, uw):
    bt, bf = 256, 1024
    return pl.pallas_call(
        _s1_body,
        grid=(FS // bf, S // bt),
        in_specs=[
            pl.BlockSpec((bt, D), lambda f, t: (t, 0)),
            pl.BlockSpec((bf, D), lambda f, t: (f, 0)),
            pl.BlockSpec((bf, D), lambda f, t: (f, 0)),
        ],
        out_specs=pl.BlockSpec((bt, bf), lambda f, t: (t, f)),
        out_shape=jax.ShapeDtypeStruct((S, FS), BF),
    )(xb, gw, uw)


def _s2_body(h_ref, dw_ref, o_ref):
    o_ref[...] = _dot_t(h_ref[...], dw_ref[...].astype(BF))


def _shared_out(h, dw):
    bt = 256
    return pl.pallas_call(
        _s2_body,
        grid=(S // bt,),
        in_specs=[
            pl.BlockSpec((bt, FS), lambda t: (t, 0)),
            pl.BlockSpec((D, FS), lambda t: (0, 0)),
        ],
        out_specs=pl.BlockSpec((bt, D), lambda t: (t, 0)),
        out_shape=jax.ShapeDtypeStruct((S, D), F32),
    )(h, dw)


def _g1_body(te_ref, ob_ref, tv_ref, xg_ref, w1_ref, w3_ref, h_ref):
    t = pl.program_id(0)

    @pl.when(tv_ref[t] == 1)
    def _():
        xb = xg_ref[...]
        a = _dot_t(xb, w1_ref[0].astype(BF))
        b = _dot_t(xb, w3_ref[0].astype(BF))
        h_ref[...] = (a * jax.nn.sigmoid(a) * b).astype(BF)


def _grouped_h(xg, w1, w3, te, ob, tv):
    spec = pltpu.PrefetchScalarGridSpec(
        num_scalar_prefetch=3,
        grid=(MAXB,),
        in_specs=[
            pl.BlockSpec((T, D), lambda t, te, ob, tv: (t, 0)),
            pl.BlockSpec((1, F, D), lambda t, te, ob, tv: (te[t], 0, 0)),
            pl.BlockSpec((1, F, D), lambda t, te, ob, tv: (te[t], 0, 0)),
        ],
        out_specs=pl.BlockSpec((T, F), lambda t, te, ob, tv: (t, 0)),
    )
    return pl.pallas_call(
        _g1_body,
        grid_spec=spec,
        out_shape=jax.ShapeDtypeStruct((P, F), BF),
    )(te, ob, tv, xg, w1, w3)


def _g2_body(te_ref, ob_ref, tv_ref, h_ref, w2_ref, ws_ref, y_ref):
    t = pl.program_id(0)

    @pl.when(tv_ref[t] == 1)
    def _():
        y = _dot_t(h_ref[...], w2_ref[0].astype(BF))
        y_ref[...] = (y * ws_ref[0, 0][:, None]).astype(BF)


def _grouped_out(h, w2, ws3d, te, ob, tv):
    spec = pltpu.PrefetchScalarGridSpec(
        num_scalar_prefetch=3,
        grid=(MAXB,),
        in_specs=[
            pl.BlockSpec((T, F), lambda t, te, ob, tv: (t, 0)),
            pl.BlockSpec((1, D, F), lambda t, te, ob, tv: (te[t], 0, 0)),
            pl.BlockSpec((1, 1, T), lambda t, te, ob, tv: (t, 0, 0)),
        ],
        out_specs=pl.BlockSpec((T, D), lambda t, te, ob, tv: (ob[t], 0)),
    )
    return pl.pallas_call(
        _g2_body,
        grid_spec=spec,
        out_shape=jax.ShapeDtypeStruct((POUT, D), BF),
    )(te, ob, tv, h, w2, ws3d)


def _dispatch_plan(e1, e2, wA, wB):
    ef = jnp.stack([e1, e2], axis=1).reshape(-1)
    wf = jnp.stack([wA, wB], axis=1).reshape(-1)
    oh = (ef[:, None] == jnp.arange(E)[None, :]).astype(jnp.int32)
    ranks = jnp.cumsum(oh, axis=0)
    rank = jnp.sum(ranks * oh, axis=1) - 1
    counts = ranks[-1]
    nblk = (counts + T - 1) // T
    ends = jnp.cumsum(nblk)
    startblk = ends - nblk
    ppos = startblk[ef] * T + rank
    tok = jnp.arange(2 * S, dtype=jnp.int32) // 2
    disp = jnp.zeros((P,), jnp.int32).at[ppos].set(
        tok, mode="drop", unique_indices=True)
    ws = jnp.zeros((P,), F32).at[ppos].set(
        wf, mode="drop", unique_indices=True)
    p0 = ppos[0::2]
    p1 = ppos[1::2]
    tvec = jnp.arange(MAXB, dtype=jnp.int32)
    used = ends[-1]
    te = jnp.minimum(jnp.searchsorted(ends, tvec, side="right"),
                     E - 1).astype(jnp.int32)
    tv = (tvec < used).astype(jnp.int32)
    ob = jnp.where(tv == 1, tvec, MAXB).astype(jnp.int32)
    return disp, ws, p0, p1, te, tv, ob


def kernel(hidden_states, wg, shared_gate_w, shared_up_w, shared_down_w, w1, w2, w3):
    B = hidden_states.shape[0]
    x = hidden_states.reshape(S, D)
    xb = x.astype(BF)

    e1, e2, wA, wB = _router(xb, wg)
    disp, ws, p0, p1, te, tv, ob = _dispatch_plan(e1, e2, wA, wB)

    hs = _shared_h(xb, shared_gate_w, shared_up_w)
    shared = _shared_out(hs, shared_down_w)

    xg = jnp.take(xb, disp, axis=0)
    h = _grouped_h(xg, w1, w3, te, ob, tv)
    y = _grouped_out(h, w2, ws.reshape(MAXB, 1, T), te, ob, tv)

    routed = (jnp.take(y, p0, axis=0).astype(F32)
              + jnp.take(y, p1, axis=0).astype(F32))
    return (shared + routed).reshape(B, S, D)


# speedup vs baseline: 1.1099x; 1.0894x over previous
import functools

import jax
import jax.numpy as jnp
from jax import lax
from jax.experimental import pallas as pl
from jax.experimental.pallas import tpu as pltpu
from jax.experimental.pallas import tpu_sc as plsc

S, D, E, F, FS = 2048, 2048, 8, 1024, 2048
T = 256
MAXB = (S * 2) // T + E - 1
P = MAXB * T
POUT = (MAXB + 1) * T

BF = jnp.bfloat16
F32 = jnp.float32


def _dot_t(a, b):
    return jax.lax.dot_general(a, b, (((1,), (1,)), ((), ())),
                               preferred_element_type=F32)


def _router_body(x_ref, wg_ref, e1_ref, e2_ref, w1_ref, w2_ref):
    logits = _dot_t(x_ref[...], wg_ref[...].astype(BF))
    m = jnp.max(logits, axis=-1, keepdims=True)
    ex = jnp.exp(logits - m)
    g = ex / jnp.sum(ex, axis=-1, keepdims=True)
    iota8 = jax.lax.broadcasted_iota(jnp.int32, g.shape, 1)
    g1 = jnp.max(g, axis=-1)
    i1 = jnp.min(jnp.where(g == g1[:, None], iota8, E), axis=-1)
    gm = jnp.where(iota8 == i1[:, None], -jnp.inf, g)
    g2 = jnp.max(gm, axis=-1)
    i2 = jnp.min(jnp.where(gm == g2[:, None], iota8, E), axis=-1)
    s = g1 + g2
    e1_ref[...] = i1
    e2_ref[...] = i2
    w1_ref[...] = g1 / s
    w2_ref[...] = g2 / s


def _router(xb, wg):
    bt = 256
    return pl.pallas_call(
        _router_body,
        grid=(S // bt,),
        in_specs=[
            pl.BlockSpec((bt, D), lambda t: (t, 0)),
            pl.BlockSpec((E, D), lambda t: (0, 0)),
        ],
        out_specs=[
            pl.BlockSpec((bt,), lambda t: (t,)),
            pl.BlockSpec((bt,), lambda t: (t,)),
            pl.BlockSpec((bt,), lambda t: (t,)),
            pl.BlockSpec((bt,), lambda t: (t,)),
        ],
        out_shape=[
            jax.ShapeDtypeStruct((S,), jnp.int32),
            jax.ShapeDtypeStruct((S,), jnp.int32),
            jax.ShapeDtypeStruct((S,), F32),
            jax.ShapeDtypeStruct((S,), F32),
        ],
    )(xb, wg)


def _s1_body(x_ref, gw_ref, uw_ref, h_ref):
    a = _dot_t(x_ref[...], gw_ref[...].astype(BF))
    b = _dot_t(x_ref[...], uw_ref[...].astype(BF))
    h_ref[...] = (a * jax.nn.sigmoid(a) * b).astype(BF)


def _shared_h(xb, gw, uw):
    bt, bf = 256, 1024
    return pl.pallas_call(
        _s1_body,
        grid=(FS // bf, S // bt),
        in_specs=[
            pl.BlockSpec((bt, D), lambda f, t: (t, 0)),
            pl.BlockSpec((bf, D), lambda f, t: (f, 0)),
            pl.BlockSpec((bf, D), lambda f, t: (f, 0)),
        ],
        out_specs=pl.BlockSpec((bt, bf), lambda f, t: (t, f)),
        out_shape=jax.ShapeDtypeStruct((S, FS), BF),
    )(xb, gw, uw)


def _s2_body(h_ref, dw_ref, o_ref):
    o_ref[...] = _dot_t(h_ref[...], dw_ref[...].astype(BF))


def _shared_out(h, dw):
    bt = 256
    return pl.pallas_call(
        _s2_body,
        grid=(S // bt,),
        in_specs=[
            pl.BlockSpec((bt, FS), lambda t: (t, 0)),
            pl.BlockSpec((D, FS), lambda t: (0, 0)),
        ],
        out_specs=pl.BlockSpec((bt, D), lambda t: (t, 0)),
        out_shape=jax.ShapeDtypeStruct((S, D), F32),
    )(h, dw)


def _g1_body(te_ref, ob_ref, tv_ref, xg_ref, w1_ref, w3_ref, h_ref):
    t = pl.program_id(0)

    @pl.when(tv_ref[t] == 1)
    def _():
        xb = xg_ref[...].astype(BF)
        a = _dot_t(xb, w1_ref[0].astype(BF))
        b = _dot_t(xb, w3_ref[0].astype(BF))
        h_ref[...] = (a * jax.nn.sigmoid(a) * b).astype(BF)


def _grouped_h(xg, w1, w3, te, ob, tv):
    spec = pltpu.PrefetchScalarGridSpec(
        num_scalar_prefetch=3,
        grid=(MAXB,),
        in_specs=[
            pl.BlockSpec((T, D), lambda t, te, ob, tv: (t, 0)),
            pl.BlockSpec((1, F, D), lambda t, te, ob, tv: (te[t], 0, 0)),
            pl.BlockSpec((1, F, D), lambda t, te, ob, tv: (te[t], 0, 0)),
        ],
        out_specs=pl.BlockSpec((T, F), lambda t, te, ob, tv: (t, 0)),
    )
    return pl.pallas_call(
        _g1_body,
        grid_spec=spec,
        out_shape=jax.ShapeDtypeStruct((P, F), BF),
    )(te, ob, tv, xg, w1, w3)


def _g2_body(te_ref, ob_ref, tv_ref, h_ref, w2_ref, ws_ref, y_ref):
    t = pl.program_id(0)

    @pl.when(tv_ref[t] == 1)
    def _():
        y = _dot_t(h_ref[...], w2_ref[0].astype(BF))
        y_ref[...] = y * ws_ref[0, 0][:, None]


def _grouped_out(h, w2, ws3d, te, ob, tv):
    spec = pltpu.PrefetchScalarGridSpec(
        num_scalar_prefetch=3,
        grid=(MAXB,),
        in_specs=[
            pl.BlockSpec((T, F), lambda t, te, ob, tv: (t, 0)),
            pl.BlockSpec((1, D, F), lambda t, te, ob, tv: (te[t], 0, 0)),
            pl.BlockSpec((1, 1, T), lambda t, te, ob, tv: (t, 0, 0)),
        ],
        out_specs=pl.BlockSpec((T, D), lambda t, te, ob, tv: (ob[t], 0)),
    )
    return pl.pallas_call(
        _g2_body,
        grid_spec=spec,
        out_shape=jax.ShapeDtypeStruct((POUT, D), F32),
    )(te, ob, tv, h, w2, ws3d)


_NC, _NS, _L = 2, 16, 16
_NW = _NC * _NS
_PAIRS = 2 * S
_PERW = _PAIRS // _NW
_DCH = 16


def _sc_dispatch(x, p0, p1):
    mesh = plsc.VectorSubcoreMesh(core_axis_name="c", subcore_axis_name="s")

    @functools.partial(
        pl.kernel, mesh=mesh,
        out_type=jax.ShapeDtypeStruct((P, D), F32),
        scratch_types=[
            pltpu.VMEM((_DCH, D), F32),
            pltpu.VMEM((_DCH,), jnp.int32),
            pltpu.VMEM((_DCH,), jnp.int32),
            pltpu.SemaphoreType.DMA,
            pltpu.SemaphoreType.DMA,
        ],
    )
    def k(x_hbm, p0_hbm, p1_hbm, xg_hbm, rows_v, i0_v, i1_v, gsem, ssem):
        wid = lax.axis_index("s") * _NC + lax.axis_index("c")
        base = wid * (S // _NW)
        for c in range((S // _NW) // _DCH):
            t0 = base + c * _DCH
            pltpu.sync_copy(p0_hbm.at[pl.ds(t0, _DCH)], i0_v)
            pltpu.sync_copy(p1_hbm.at[pl.ds(t0, _DCH)], i1_v)
            pltpu.async_copy(x_hbm.at[pl.ds(t0, _DCH)], rows_v, gsem).wait()
            pltpu.async_copy(rows_v, xg_hbm.at[i0_v], ssem).wait()
            pltpu.async_copy(rows_v, xg_hbm.at[i1_v], ssem).wait()

    return k(x, p0, p1)


_TPW = S // _NW
_CCH = 8


def _sc_combine(shared, y, p0, p1):
    mesh = plsc.VectorSubcoreMesh(core_axis_name="c", subcore_axis_name="s")

    @functools.partial(
        pl.kernel, mesh=mesh,
        out_type=jax.ShapeDtypeStruct((S, D), F32),
        scratch_types=[
            pltpu.VMEM((_CCH, D), F32),
            pltpu.VMEM((_CCH, D), F32),
            pltpu.VMEM((_CCH, D), F32),
            pltpu.VMEM((_CCH,), jnp.int32),
            pltpu.VMEM((_CCH,), jnp.int32),
            pltpu.SemaphoreType.DMA,
            pltpu.SemaphoreType.DMA,
        ],
    )
    def k(sh_hbm, y_hbm, p0_hbm, p1_hbm, out_hbm,
          sh_v, y0_v, y1_v, i0_v, i1_v, sem0, sem1):
        wid = lax.axis_index("s") * _NC + lax.axis_index("c")
        base = wid * _TPW
        for c in range(_TPW // _CCH):
            t0 = base + c * _CCH
            pltpu.sync_copy(p0_hbm.at[pl.ds(t0, _CCH)], i0_v)
            pltpu.sync_copy(p1_hbm.at[pl.ds(t0, _CCH)], i1_v)
            cp0 = pltpu.async_copy(y_hbm.at[i0_v], y0_v, sem0)
            cp1 = pltpu.async_copy(y_hbm.at[i1_v], y1_v, sem1)
            pltpu.sync_copy(sh_hbm.at[pl.ds(t0, _CCH)], sh_v)
            cp0.wait()
            cp1.wait()

            def row(r, _):
                def col(kk, _2):
                    sl = pl.ds(kk * _L, _L)
                    sh_v[r, sl] = sh_v[r, sl] + y0_v[r, sl] + y1_v[r, sl]
                    return 0

                lax.fori_loop(0, D // _L, col, 0)
                return 0

            lax.fori_loop(0, _CCH, row, 0)
            pltpu.sync_copy(sh_v, out_hbm.at[pl.ds(t0, _CCH)])

    return k(shared, y, p0, p1)


def _dispatch_plan(e1, e2, wA, wB):
    ef = jnp.stack([e1, e2], axis=1).reshape(-1)
    oh = (ef[:, None] == jnp.arange(E)[None, :]).astype(jnp.int32)
    ranks = jnp.cumsum(oh, axis=0)
    rank = jnp.sum(ranks * oh, axis=1) - 1
    counts = ranks[-1]
    nblk = (counts + T - 1) // T
    ends = jnp.cumsum(nblk)
    startblk = ends - nblk
    ppos = (startblk[ef] * T + rank).astype(jnp.int32)
    wf = jnp.stack([wA, wB], axis=1).reshape(-1)
    ws = jnp.zeros((P,), F32).at[ppos].set(
        wf, mode="drop", unique_indices=True)
    p0 = ppos[0::2]
    p1 = ppos[1::2]
    tvec = jnp.arange(MAXB, dtype=jnp.int32)
    used = ends[-1]
    te = jnp.minimum(jnp.searchsorted(ends, tvec, side="right"),
                     E - 1).astype(jnp.int32)
    tv = (tvec < used).astype(jnp.int32)
    ob = jnp.where(tv == 1, tvec, MAXB).astype(jnp.int32)
    return ws, p0, p1, te, tv, ob


def kernel(hidden_states, wg, shared_gate_w, shared_up_w, shared_down_w, w1, w2, w3):
    B = hidden_states.shape[0]
    x = hidden_states.reshape(S, D)
    xb = x.astype(BF)

    e1, e2, wfa, wfb = _router(xb, wg)
    ws, p0, p1, te, tv, ob = _dispatch_plan(e1, e2, wfa, wfb)

    xg = _sc_dispatch(x, p0, p1)

    hs = _shared_h(xb, shared_gate_w, shared_up_w)
    shared = _shared_out(hs, shared_down_w)

    h = _grouped_h(xg, w1, w3, te, ob, tv)
    y = _grouped_out(h, w2, ws.reshape(MAXB, 1, T), te, ob, tv)

    return _sc_combine(shared, y, p0, p1).reshape(B, S, D)


# speedup vs baseline: 1.1366x; 1.0240x over previous
import functools

import jax
import jax.numpy as jnp
from jax import lax
from jax.experimental import pallas as pl
from jax.experimental.pallas import tpu as pltpu
from jax.experimental.pallas import tpu_sc as plsc

S, D, E, F, FS = 2048, 2048, 8, 1024, 2048
T = 256
MAXB = (S * 2) // T + E - 1
P = MAXB * T
POUT = (MAXB + 1) * T

BF = jnp.bfloat16
F32 = jnp.float32


def _dot_t(a, b):
    return jax.lax.dot_general(a, b, (((1,), (1,)), ((), ())),
                               preferred_element_type=F32)


def _router_body(x_ref, wg_ref, e1_ref, e2_ref, w1_ref, w2_ref):
    logits = _dot_t(x_ref[...], wg_ref[...].astype(BF))
    m = jnp.max(logits, axis=-1, keepdims=True)
    ex = jnp.exp(logits - m)
    g = ex / jnp.sum(ex, axis=-1, keepdims=True)
    iota8 = jax.lax.broadcasted_iota(jnp.int32, g.shape, 1)
    g1 = jnp.max(g, axis=-1)
    i1 = jnp.min(jnp.where(g == g1[:, None], iota8, E), axis=-1)
    gm = jnp.where(iota8 == i1[:, None], -jnp.inf, g)
    g2 = jnp.max(gm, axis=-1)
    i2 = jnp.min(jnp.where(gm == g2[:, None], iota8, E), axis=-1)
    s = g1 + g2
    e1_ref[...] = i1
    e2_ref[...] = i2
    w1_ref[...] = g1 / s
    w2_ref[...] = g2 / s


def _router(xb, wg):
    bt = 256
    return pl.pallas_call(
        _router_body,
        grid=(S // bt,),
        in_specs=[
            pl.BlockSpec((bt, D), lambda t: (t, 0)),
            pl.BlockSpec((E, D), lambda t: (0, 0)),
        ],
        out_specs=[
            pl.BlockSpec((bt,), lambda t: (t,)),
            pl.BlockSpec((bt,), lambda t: (t,)),
            pl.BlockSpec((bt,), lambda t: (t,)),
            pl.BlockSpec((bt,), lambda t: (t,)),
        ],
        out_shape=[
            jax.ShapeDtypeStruct((S,), jnp.int32),
            jax.ShapeDtypeStruct((S,), jnp.int32),
            jax.ShapeDtypeStruct((S,), F32),
            jax.ShapeDtypeStruct((S,), F32),
        ],
    )(xb, wg)


def _s1_body(x_ref, gw_ref, uw_ref, h_ref):
    a = _dot_t(x_ref[...], gw_ref[...].astype(BF))
    b = _dot_t(x_ref[...], uw_ref[...].astype(BF))
    h_ref[...] = (a * jax.nn.sigmoid(a) * b).astype(BF)


def _shared_h(xb, gw, uw):
    bt, bf = 512, 1024
    return pl.pallas_call(
        _s1_body,
        grid=(FS // bf, S // bt),
        in_specs=[
            pl.BlockSpec((bt, D), lambda f, t: (t, 0)),
            pl.BlockSpec((bf, D), lambda f, t: (f, 0)),
            pl.BlockSpec((bf, D), lambda f, t: (f, 0)),
        ],
        out_specs=pl.BlockSpec((bt, bf), lambda f, t: (t, f)),
        out_shape=jax.ShapeDtypeStruct((S, FS), BF),
    )(xb, gw, uw)


def _s2_body(h_ref, dw_ref, o_ref):
    o_ref[...] = _dot_t(h_ref[...], dw_ref[...].astype(BF))


def _shared_out(h, dw):
    bt = 512
    return pl.pallas_call(
        _s2_body,
        grid=(S // bt,),
        in_specs=[
            pl.BlockSpec((bt, FS), lambda t: (t, 0)),
            pl.BlockSpec((D, FS), lambda t: (0, 0)),
        ],
        out_specs=pl.BlockSpec((bt, D), lambda t: (t, 0)),
        out_shape=jax.ShapeDtypeStruct((S, D), F32),
    )(h, dw)


def _g1_body(te_ref, ob_ref, tv_ref, xg_ref, w1_ref, w3_ref, h_ref):
    t = pl.program_id(0)

    @pl.when(tv_ref[t] == 1)
    def _():
        xb = xg_ref[...].astype(BF)
        a = _dot_t(xb, w1_ref[0].astype(BF))
        b = _dot_t(xb, w3_ref[0].astype(BF))
        h_ref[...] = (a * jax.nn.sigmoid(a) * b).astype(BF)


def _grouped_h(xg, w1, w3, te, ob, tv):
    spec = pltpu.PrefetchScalarGridSpec(
        num_scalar_prefetch=3,
        grid=(MAXB,),
        in_specs=[
            pl.BlockSpec((T, D), lambda t, te, ob, tv: (t, 0)),
            pl.BlockSpec((1, F, D), lambda t, te, ob, tv: (te[t], 0, 0)),
            pl.BlockSpec((1, F, D), lambda t, te, ob, tv: (te[t], 0, 0)),
        ],
        out_specs=pl.BlockSpec((T, F), lambda t, te, ob, tv: (t, 0)),
    )
    return pl.pallas_call(
        _g1_body,
        grid_spec=spec,
        out_shape=jax.ShapeDtypeStruct((P, F), BF),
    )(te, ob, tv, xg, w1, w3)


def _g2_body(te_ref, ob_ref, tv_ref, h_ref, w2_ref, ws_ref, y_ref):
    t = pl.program_id(0)

    @pl.when(tv_ref[t] == 1)
    def _():
        y = _dot_t(h_ref[...], w2_ref[0].astype(BF))
        y_ref[...] = y * ws_ref[0, 0][:, None]


def _grouped_out(h, w2, ws3d, te, ob, tv):
    spec = pltpu.PrefetchScalarGridSpec(
        num_scalar_prefetch=3,
        grid=(MAXB,),
        in_specs=[
            pl.BlockSpec((T, F), lambda t, te, ob, tv: (t, 0)),
            pl.BlockSpec((1, D, F), lambda t, te, ob, tv: (te[t], 0, 0)),
            pl.BlockSpec((1, 1, T), lambda t, te, ob, tv: (t, 0, 0)),
        ],
        out_specs=pl.BlockSpec((T, D), lambda t, te, ob, tv: (ob[t], 0)),
    )
    return pl.pallas_call(
        _g2_body,
        grid_spec=spec,
        out_shape=jax.ShapeDtypeStruct((POUT, D), F32),
    )(te, ob, tv, h, w2, ws3d)


_NC, _NS, _L = 2, 16, 16
_NW = _NC * _NS
_PAIRS = 2 * S
_PERW = _PAIRS // _NW
_DCH = 16


def _sc_dispatch(x, p0, p1):
    mesh = plsc.VectorSubcoreMesh(core_axis_name="c", subcore_axis_name="s")

    @functools.partial(
        pl.kernel, mesh=mesh,
        out_type=jax.ShapeDtypeStruct((P, D), F32),
        scratch_types=[
            pltpu.VMEM((_DCH, D), F32),
            pltpu.VMEM((_DCH,), jnp.int32),
            pltpu.VMEM((_DCH,), jnp.int32),
            pltpu.SemaphoreType.DMA,
            pltpu.SemaphoreType.DMA,
        ],
    )
    def k(x_hbm, p0_hbm, p1_hbm, xg_hbm, rows_v, i0_v, i1_v, gsem, ssem):
        wid = lax.axis_index("s") * _NC + lax.axis_index("c")
        base = wid * (S // _NW)
        for c in range((S // _NW) // _DCH):
            t0 = base + c * _DCH
            pltpu.sync_copy(p0_hbm.at[pl.ds(t0, _DCH)], i0_v)
            pltpu.sync_copy(p1_hbm.at[pl.ds(t0, _DCH)], i1_v)
            pltpu.async_copy(x_hbm.at[pl.ds(t0, _DCH)], rows_v, gsem).wait()
            pltpu.async_copy(rows_v, xg_hbm.at[i0_v], ssem).wait()
            pltpu.async_copy(rows_v, xg_hbm.at[i1_v], ssem).wait()

    return k(x, p0, p1)


_TPW = S // _NW
_CCH = 8


def _sc_combine(shared, y, p0, p1):
    mesh = plsc.VectorSubcoreMesh(core_axis_name="c", subcore_axis_name="s")

    @functools.partial(
        pl.kernel, mesh=mesh,
        out_type=jax.ShapeDtypeStruct((S, D), F32),
        scratch_types=[
            pltpu.VMEM((_CCH, D), F32),
            pltpu.VMEM((_CCH, D), F32),
            pltpu.VMEM((_CCH, D), F32),
            pltpu.VMEM((_CCH,), jnp.int32),
            pltpu.VMEM((_CCH,), jnp.int32),
            pltpu.SemaphoreType.DMA,
            pltpu.SemaphoreType.DMA,
        ],
    )
    def k(sh_hbm, y_hbm, p0_hbm, p1_hbm, out_hbm,
          sh_v, y0_v, y1_v, i0_v, i1_v, sem0, sem1):
        wid = lax.axis_index("s") * _NC + lax.axis_index("c")
        base = wid * _TPW

        def chunk(c, _):
            t0 = pl.multiple_of(base + c * _CCH, 8)
            pltpu.sync_copy(p0_hbm.at[pl.ds(t0, _CCH)], i0_v)
            pltpu.sync_copy(p1_hbm.at[pl.ds(t0, _CCH)], i1_v)
            cp0 = pltpu.async_copy(y_hbm.at[i0_v], y0_v, sem0)
            cp1 = pltpu.async_copy(y_hbm.at[i1_v], y1_v, sem1)
            pltpu.sync_copy(sh_hbm.at[pl.ds(t0, _CCH)], sh_v)
            cp0.wait()
            cp1.wait()

            def row(r, _2):
                for kk in range(D // _L):
                    sl = pl.ds(kk * _L, _L)
                    sh_v[r, sl] = sh_v[r, sl] + y0_v[r, sl] + y1_v[r, sl]
                return 0

            lax.fori_loop(0, _CCH, row, 0)
            pltpu.sync_copy(sh_v, out_hbm.at[pl.ds(t0, _CCH)])
            return 0

        lax.fori_loop(0, _TPW // _CCH, chunk, 0)

    return k(shared, y, p0, p1)


def _dispatch_plan(e1, e2, wA, wB):
    ef = jnp.stack([e1, e2], axis=1).reshape(-1)
    oh = (ef[:, None] == jnp.arange(E)[None, :]).astype(jnp.int32)
    ranks = jnp.cumsum(oh, axis=0)
    rank = jnp.sum(ranks * oh, axis=1) - 1
    counts = ranks[-1]
    nblk = (counts + T - 1) // T
    ends = jnp.cumsum(nblk)
    startblk = ends - nblk
    ppos = (startblk[ef] * T + rank).astype(jnp.int32)
    wf = jnp.stack([wA, wB], axis=1).reshape(-1)
    ws = jnp.zeros((P,), F32).at[ppos].set(
        wf, mode="drop", unique_indices=True)
    p0 = ppos[0::2]
    p1 = ppos[1::2]
    tvec = jnp.arange(MAXB, dtype=jnp.int32)
    used = ends[-1]
    te = jnp.minimum(jnp.searchsorted(ends, tvec, side="right"),
                     E - 1).astype(jnp.int32)
    tv = (tvec < used).astype(jnp.int32)
    ob = jnp.where(tv == 1, tvec, MAXB).astype(jnp.int32)
    return ws, p0, p1, te, tv, ob


def kernel(hidden_states, wg, shared_gate_w, shared_up_w, shared_down_w, w1, w2, w3):
    B = hidden_states.shape[0]
    x = hidden_states.reshape(S, D)
    xb = x.astype(BF)

    e1, e2, wfa, wfb = _router(xb, wg)
    ws, p0, p1, te, tv, ob = _dispatch_plan(e1, e2, wfa, wfb)

    xg = _sc_dispatch(x, p0, p1)

    hs = _shared_h(xb, shared_gate_w, shared_up_w)
    shared = _shared_out(hs, shared_down_w)

    h = _grouped_h(xg, w1, w3, te, ob, tv)
    y = _grouped_out(h, w2, ws.reshape(MAXB, 1, T), te, ob, tv)

    return _sc_combine(shared, y, p0, p1).reshape(B, S, D)


# speedup vs baseline: 1.1601x; 1.0207x over previous
import functools

import jax
import jax.numpy as jnp
from jax import lax
from jax.experimental import pallas as pl
from jax.experimental.pallas import tpu as pltpu
from jax.experimental.pallas import tpu_sc as plsc

S, D, E, F, FS = 2048, 2048, 8, 1024, 2048
T = 256
MAXB = (S * 2) // T + E - 1
P = MAXB * T
POUT = (MAXB + 1) * T

BF = jnp.bfloat16
F32 = jnp.float32


def _dot_t(a, b):
    return jax.lax.dot_general(a, b, (((1,), (1,)), ((), ())),
                               preferred_element_type=F32)


def _router_body(x_ref, wg_ref, e1_ref, e2_ref, w1_ref, w2_ref):
    logits = _dot_t(x_ref[...], wg_ref[...].astype(BF))
    m = jnp.max(logits, axis=-1, keepdims=True)
    ex = jnp.exp(logits - m)
    g = ex / jnp.sum(ex, axis=-1, keepdims=True)
    iota8 = jax.lax.broadcasted_iota(jnp.int32, g.shape, 1)
    g1 = jnp.max(g, axis=-1)
    i1 = jnp.min(jnp.where(g == g1[:, None], iota8, E), axis=-1)
    gm = jnp.where(iota8 == i1[:, None], -jnp.inf, g)
    g2 = jnp.max(gm, axis=-1)
    i2 = jnp.min(jnp.where(gm == g2[:, None], iota8, E), axis=-1)
    s = g1 + g2
    e1_ref[...] = i1
    e2_ref[...] = i2
    w1_ref[...] = g1 / s
    w2_ref[...] = g2 / s


def _router(xb, wg):
    bt = 256
    return pl.pallas_call(
        _router_body,
        grid=(S // bt,),
        in_specs=[
            pl.BlockSpec((bt, D), lambda t: (t, 0)),
            pl.BlockSpec((E, D), lambda t: (0, 0)),
        ],
        out_specs=[
            pl.BlockSpec((bt,), lambda t: (t,)),
            pl.BlockSpec((bt,), lambda t: (t,)),
            pl.BlockSpec((bt,), lambda t: (t,)),
            pl.BlockSpec((bt,), lambda t: (t,)),
        ],
        out_shape=[
            jax.ShapeDtypeStruct((S,), jnp.int32),
            jax.ShapeDtypeStruct((S,), jnp.int32),
            jax.ShapeDtypeStruct((S,), F32),
            jax.ShapeDtypeStruct((S,), F32),
        ],
    )(xb, wg)


def _s1_body(x_ref, gw_ref, uw_ref, h_ref):
    a = _dot_t(x_ref[...], gw_ref[...].astype(BF))
    b = _dot_t(x_ref[...], uw_ref[...].astype(BF))
    h_ref[...] = (a * jax.nn.sigmoid(a) * b).astype(BF)


def _shared_h(xb, gw, uw):
    bt, bf = 512, 1024
    return pl.pallas_call(
        _s1_body,
        grid=(FS // bf, S // bt),
        in_specs=[
            pl.BlockSpec((bt, D), lambda f, t: (t, 0)),
            pl.BlockSpec((bf, D), lambda f, t: (f, 0)),
            pl.BlockSpec((bf, D), lambda f, t: (f, 0)),
        ],
        out_specs=pl.BlockSpec((bt, bf), lambda f, t: (t, f)),
        out_shape=jax.ShapeDtypeStruct((S, FS), BF),
    )(xb, gw, uw)


def _s2_body(h_ref, dw_ref, o_ref):
    o_ref[...] = _dot_t(h_ref[...], dw_ref[...].astype(BF))


def _shared_out(h, dw):
    bt = 512
    return pl.pallas_call(
        _s2_body,
        grid=(S // bt,),
        in_specs=[
            pl.BlockSpec((bt, FS), lambda t: (t, 0)),
            pl.BlockSpec((D, FS), lambda t: (0, 0)),
        ],
        out_specs=pl.BlockSpec((bt, D), lambda t: (t, 0)),
        out_shape=jax.ShapeDtypeStruct((S, D), F32),
    )(h, dw)


def _g1_body(te_ref, ob_ref, tv_ref, xg_ref, w1_ref, w3_ref, h_ref):
    t = pl.program_id(0)

    @pl.when(tv_ref[t] == 1)
    def _():
        xb = xg_ref[...].astype(BF)
        a = _dot_t(xb, w1_ref[0].astype(BF))
        b = _dot_t(xb, w3_ref[0].astype(BF))
        h_ref[...] = (a * jax.nn.sigmoid(a) * b).astype(BF)


def _grouped_h(xg, w1, w3, te, ob, tv):
    spec = pltpu.PrefetchScalarGridSpec(
        num_scalar_prefetch=3,
        grid=(MAXB,),
        in_specs=[
            pl.BlockSpec((T, D), lambda t, te, ob, tv: (t, 0)),
            pl.BlockSpec((1, F, D), lambda t, te, ob, tv: (te[t], 0, 0)),
            pl.BlockSpec((1, F, D), lambda t, te, ob, tv: (te[t], 0, 0)),
        ],
        out_specs=pl.BlockSpec((T, F), lambda t, te, ob, tv: (t, 0)),
    )
    return pl.pallas_call(
        _g1_body,
        grid_spec=spec,
        out_shape=jax.ShapeDtypeStruct((P, F), BF),
    )(te, ob, tv, xg, w1, w3)


def _g2_body(te_ref, ob_ref, tv_ref, h_ref, w2_ref, ws_ref, y_ref):
    t = pl.program_id(0)

    @pl.when(tv_ref[t] == 1)
    def _():
        y = _dot_t(h_ref[...], w2_ref[0].astype(BF))
        y_ref[...] = y * ws_ref[0, 0][:, None]


def _grouped_out(h, w2, ws3d, te, ob, tv):
    spec = pltpu.PrefetchScalarGridSpec(
        num_scalar_prefetch=3,
        grid=(MAXB,),
        in_specs=[
            pl.BlockSpec((T, F), lambda t, te, ob, tv: (t, 0)),
            pl.BlockSpec((1, D, F), lambda t, te, ob, tv: (te[t], 0, 0)),
            pl.BlockSpec((1, 1, T), lambda t, te, ob, tv: (t, 0, 0)),
        ],
        out_specs=pl.BlockSpec((T, D), lambda t, te, ob, tv: (ob[t], 0)),
    )
    return pl.pallas_call(
        _g2_body,
        grid_spec=spec,
        out_shape=jax.ShapeDtypeStruct((POUT, D), F32),
    )(te, ob, tv, h, w2, ws3d)


_NC, _NS, _L = 2, 16, 16
_NW = _NC * _NS
_PAIRS = 2 * S
_PERW = _PAIRS // _NW
_DCH = 16


def _sc_dispatch(x, p0, p1):
    mesh = plsc.VectorSubcoreMesh(core_axis_name="c", subcore_axis_name="s")

    @functools.partial(
        pl.kernel, mesh=mesh,
        out_type=jax.ShapeDtypeStruct((P, D), F32),
        scratch_types=[
            pltpu.VMEM((_DCH, D), F32),
            pltpu.VMEM((_DCH,), jnp.int32),
            pltpu.VMEM((_DCH,), jnp.int32),
            pltpu.SemaphoreType.DMA,
            pltpu.SemaphoreType.DMA,
        ],
    )
    def k(x_hbm, p0_hbm, p1_hbm, xg_hbm, rows_v, i0_v, i1_v, gsem, ssem):
        wid = lax.axis_index("s") * _NC + lax.axis_index("c")
        base = wid * (S // _NW)
        for c in range((S // _NW) // _DCH):
            t0 = base + c * _DCH
            pltpu.sync_copy(p0_hbm.at[pl.ds(t0, _DCH)], i0_v)
            pltpu.sync_copy(p1_hbm.at[pl.ds(t0, _DCH)], i1_v)
            pltpu.async_copy(x_hbm.at[pl.ds(t0, _DCH)], rows_v, gsem).wait()
            pltpu.async_copy(rows_v, xg_hbm.at[i0_v], ssem).wait()
            pltpu.async_copy(rows_v, xg_hbm.at[i1_v], ssem).wait()

    return k(x, p0, p1)


_TPW = S // _NW
_CCH = 16


def _sc_combine(shared, y, p0, p1):
    mesh = plsc.VectorSubcoreMesh(core_axis_name="c", subcore_axis_name="s")

    @functools.partial(
        pl.kernel, mesh=mesh,
        out_type=jax.ShapeDtypeStruct((S, D), F32),
        scratch_types=[
            pltpu.VMEM((_CCH, D), F32),
            pltpu.VMEM((_CCH, D), F32),
            pltpu.VMEM((_CCH, D), F32),
            pltpu.VMEM((_TPW,), jnp.int32),
            pltpu.VMEM((_TPW,), jnp.int32),
            pltpu.VMEM((_CCH,), jnp.int32),
            pltpu.VMEM((_CCH,), jnp.int32),
            pltpu.SemaphoreType.DMA,
            pltpu.SemaphoreType.DMA,
            pltpu.SemaphoreType.DMA,
        ],
    )
    def k(sh_hbm, y_hbm, p0_hbm, p1_hbm, out_hbm,
          sh_v, y0_v, y1_v, i0all_v, i1all_v, i0_v, i1_v, sem0, sem1, osem):
        wid = lax.axis_index("s") * _NC + lax.axis_index("c")
        base = wid * _TPW
        pltpu.sync_copy(p0_hbm.at[pl.ds(base, _TPW)], i0all_v)
        pltpu.sync_copy(p1_hbm.at[pl.ds(base, _TPW)], i1all_v)
        ocp = None
        for c in range(_TPW // _CCH):
            t0 = base + c * _CCH
            i0_v[...] = i0all_v[pl.ds(c * _CCH, _CCH)]
            i1_v[...] = i1all_v[pl.ds(c * _CCH, _CCH)]
            cp0 = pltpu.async_copy(y_hbm.at[i0_v], y0_v, sem0)
            cp1 = pltpu.async_copy(y_hbm.at[i1_v], y1_v, sem1)
            if ocp is not None:
                ocp.wait()
            pltpu.sync_copy(sh_hbm.at[pl.ds(t0, _CCH)], sh_v)
            cp0.wait()
            cp1.wait()

            def row(r, _2):
                for kk in range(D // _L):
                    sl = pl.ds(kk * _L, _L)
                    sh_v[r, sl] = sh_v[r, sl] + y0_v[r, sl] + y1_v[r, sl]
                return 0

            lax.fori_loop(0, _CCH, row, 0)
            ocp = pltpu.async_copy(sh_v, out_hbm.at[pl.ds(t0, _CCH)], osem)
        ocp.wait()

    return k(shared, y, p0, p1)


def _dispatch_plan(e1, e2, wA, wB):
    ef = jnp.stack([e1, e2], axis=1).reshape(-1)
    oh = (ef[:, None] == jnp.arange(E)[None, :]).astype(jnp.int32)
    ranks = jnp.cumsum(oh, axis=0)
    rank = jnp.sum(ranks * oh, axis=1) - 1
    counts = ranks[-1]
    nblk = (counts + T - 1) // T
    ends = jnp.cumsum(nblk)
    startblk = ends - nblk
    ppos = (startblk[ef] * T + rank).astype(jnp.int32)
    wf = jnp.stack([wA, wB], axis=1).reshape(-1)
    ws = jnp.zeros((P,), F32).at[ppos].set(
        wf, mode="drop", unique_indices=True)
    p0 = ppos[0::2]
    p1 = ppos[1::2]
    tvec = jnp.arange(MAXB, dtype=jnp.int32)
    used = ends[-1]
    te = jnp.minimum(jnp.searchsorted(ends, tvec, side="right"),
                     E - 1).astype(jnp.int32)
    tv = (tvec < used).astype(jnp.int32)
    ob = jnp.where(tv == 1, tvec, MAXB).astype(jnp.int32)
    return ws, p0, p1, te, tv, ob


def kernel(hidden_states, wg, shared_gate_w, shared_up_w, shared_down_w, w1, w2, w3):
    B = hidden_states.shape[0]
    x = hidden_states.reshape(S, D)
    xb = x.astype(BF)

    e1, e2, wfa, wfb = _router(xb, wg)
    ws, p0, p1, te, tv, ob = _dispatch_plan(e1, e2, wfa, wfb)

    xg = _sc_dispatch(x, p0, p1)

    hs = _shared_h(xb, shared_gate_w, shared_up_w)
    shared = _shared_out(hs, shared_down_w)

    h = _grouped_h(xg, w1, w3, te, ob, tv)
    y = _grouped_out(h, w2, ws.reshape(MAXB, 1, T), te, ob, tv)

    return _sc_combine(shared, y, p0, p1).reshape(B, S, D)


# speedup vs baseline: 1.2308x; 1.0610x over previous
import functools

import jax
import jax.numpy as jnp
from jax import lax
from jax.experimental import pallas as pl
from jax.experimental.pallas import tpu as pltpu
from jax.experimental.pallas import tpu_sc as plsc

S, D, E, F, FS = 2048, 2048, 8, 1024, 2048
T = 256
MAXB = (S * 2) // T + E - 1
P = MAXB * T
POUT = (MAXB + 1) * T

BF = jnp.bfloat16
F32 = jnp.float32


def _dot_t(a, b):
    return jax.lax.dot_general(a, b, (((1,), (1,)), ((), ())),
                               preferred_element_type=F32)


def _router_body(x_ref, wg_ref, xb_ref, e1_ref, e2_ref, w1_ref, w2_ref):
    xbv = x_ref[...].astype(BF)
    xb_ref[...] = xbv
    logits = _dot_t(xbv, wg_ref[...].astype(BF))
    m = jnp.max(logits, axis=-1, keepdims=True)
    ex = jnp.exp(logits - m)
    g = ex / jnp.sum(ex, axis=-1, keepdims=True)
    iota8 = jax.lax.broadcasted_iota(jnp.int32, g.shape, 1)
    g1 = jnp.max(g, axis=-1)
    i1 = jnp.min(jnp.where(g == g1[:, None], iota8, E), axis=-1)
    gm = jnp.where(iota8 == i1[:, None], -jnp.inf, g)
    g2 = jnp.max(gm, axis=-1)
    i2 = jnp.min(jnp.where(gm == g2[:, None], iota8, E), axis=-1)
    s = g1 + g2
    e1_ref[...] = i1
    e2_ref[...] = i2
    w1_ref[...] = g1 / s
    w2_ref[...] = g2 / s


def _router(x, wg):
    bt = 256
    return pl.pallas_call(
        _router_body,
        grid=(S // bt,),
        in_specs=[
            pl.BlockSpec((bt, D), lambda t: (t, 0)),
            pl.BlockSpec((E, D), lambda t: (0, 0)),
        ],
        out_specs=[
            pl.BlockSpec((bt, D), lambda t: (t, 0)),
            pl.BlockSpec((bt,), lambda t: (t,)),
            pl.BlockSpec((bt,), lambda t: (t,)),
            pl.BlockSpec((bt,), lambda t: (t,)),
            pl.BlockSpec((bt,), lambda t: (t,)),
        ],
        out_shape=[
            jax.ShapeDtypeStruct((S, D), BF),
            jax.ShapeDtypeStruct((S,), jnp.int32),
            jax.ShapeDtypeStruct((S,), jnp.int32),
            jax.ShapeDtypeStruct((S,), F32),
            jax.ShapeDtypeStruct((S,), F32),
        ],
    )(x, wg)


def _s1_body(x_ref, gw_ref, uw_ref, h_ref):
    a = _dot_t(x_ref[...], gw_ref[...].astype(BF))
    b = _dot_t(x_ref[...], uw_ref[...].astype(BF))
    h_ref[...] = (a * jax.nn.sigmoid(a) * b).astype(BF)


def _shared_h(xb, gw, uw):
    bt, bf = 512, 1024
    return pl.pallas_call(
        _s1_body,
        grid=(FS // bf, S // bt),
        in_specs=[
            pl.BlockSpec((bt, D), lambda f, t: (t, 0)),
            pl.BlockSpec((bf, D), lambda f, t: (f, 0)),
            pl.BlockSpec((bf, D), lambda f, t: (f, 0)),
        ],
        out_specs=pl.BlockSpec((bt, bf), lambda f, t: (t, f)),
        out_shape=jax.ShapeDtypeStruct((S, FS), BF),
    )(xb, gw, uw)


def _s2_body(h_ref, dw_ref, o_ref):
    o_ref[...] = _dot_t(h_ref[...], dw_ref[...].astype(BF))


def _shared_out(h, dw):
    bt = 512
    return pl.pallas_call(
        _s2_body,
        grid=(S // bt,),
        in_specs=[
            pl.BlockSpec((bt, FS), lambda t: (t, 0)),
            pl.BlockSpec((D, FS), lambda t: (0, 0)),
        ],
        out_specs=pl.BlockSpec((bt, D), lambda t: (t, 0)),
        out_shape=jax.ShapeDtypeStruct((S, D), F32),
    )(h, dw)


def _g1_body(te_ref, ob_ref, tv_ref, xg_ref, w1_ref, w3_ref, h_ref):
    t = pl.program_id(0)

    @pl.when(tv_ref[t] == 1)
    def _():
        xb = xg_ref[...].astype(BF)
        a = _dot_t(xb, w1_ref[0].astype(BF))
        b = _dot_t(xb, w3_ref[0].astype(BF))
        h_ref[...] = (a * jax.nn.sigmoid(a) * b).astype(BF)


def _grouped_h(xg, w1, w3, te, ob, tv):
    spec = pltpu.PrefetchScalarGridSpec(
        num_scalar_prefetch=3,
        grid=(MAXB,),
        in_specs=[
            pl.BlockSpec((T, D), lambda t, te, ob, tv: (t, 0)),
            pl.BlockSpec((1, F, D), lambda t, te, ob, tv: (te[t], 0, 0)),
            pl.BlockSpec((1, F, D), lambda t, te, ob, tv: (te[t], 0, 0)),
        ],
        out_specs=pl.BlockSpec((T, F), lambda t, te, ob, tv: (t, 0)),
    )
    return pl.pallas_call(
        _g1_body,
        grid_spec=spec,
        out_shape=jax.ShapeDtypeStruct((P, F), BF),
    )(te, ob, tv, xg, w1, w3)


def _g2_body(te_ref, ob_ref, tv_ref, h_ref, w2_ref, y_ref):
    t = pl.program_id(0)

    @pl.when(tv_ref[t] == 1)
    def _():
        y_ref[...] = _dot_t(h_ref[...], w2_ref[0].astype(BF))


def _grouped_out(h, w2, te, ob, tv):
    spec = pltpu.PrefetchScalarGridSpec(
        num_scalar_prefetch=3,
        grid=(MAXB,),
        in_specs=[
            pl.BlockSpec((T, F), lambda t, te, ob, tv: (t, 0)),
            pl.BlockSpec((1, D, F), lambda t, te, ob, tv: (te[t], 0, 0)),
        ],
        out_specs=pl.BlockSpec((T, D), lambda t, te, ob, tv: (ob[t], 0)),
    )
    return pl.pallas_call(
        _g2_body,
        grid_spec=spec,
        out_shape=jax.ShapeDtypeStruct((POUT, D), F32),
    )(te, ob, tv, h, w2)


_NC, _NS, _L = 2, 16, 16
_NW = _NC * _NS
_PAIRS = 2 * S
_PERW = _PAIRS // _NW
_DCH = 16


def _sc_dispatch(x, p0, p1):
    mesh = plsc.VectorSubcoreMesh(core_axis_name="c", subcore_axis_name="s")

    @functools.partial(
        pl.kernel, mesh=mesh,
        out_type=jax.ShapeDtypeStruct((P, D), F32),
        scratch_types=[
            pltpu.VMEM((_DCH, D), F32),
            pltpu.VMEM((_DCH,), jnp.int32),
            pltpu.VMEM((_DCH,), jnp.int32),
            pltpu.SemaphoreType.DMA,
            pltpu.SemaphoreType.DMA,
        ],
    )
    def k(x_hbm, p0_hbm, p1_hbm, xg_hbm, rows_v, i0_v, i1_v, gsem, ssem):
        wid = lax.axis_index("s") * _NC + lax.axis_index("c")
        base = wid * (S // _NW)
        for c in range((S // _NW) // _DCH):
            t0 = base + c * _DCH
            pltpu.sync_copy(p0_hbm.at[pl.ds(t0, _DCH)], i0_v)
            pltpu.sync_copy(p1_hbm.at[pl.ds(t0, _DCH)], i1_v)
            pltpu.async_copy(x_hbm.at[pl.ds(t0, _DCH)], rows_v, gsem).wait()
            pltpu.async_copy(rows_v, xg_hbm.at[i0_v], ssem).wait()
            pltpu.async_copy(rows_v, xg_hbm.at[i1_v], ssem).wait()

    return k(x, p0, p1)


_TPW = S // _NW
_CCH = 16


def _sc_combine(shared, y, p0, p1, wae, wbe):
    mesh = plsc.VectorSubcoreMesh(core_axis_name="c", subcore_axis_name="s")

    @functools.partial(
        pl.kernel, mesh=mesh,
        out_type=jax.ShapeDtypeStruct((S, D), F32),
        scratch_types=[
            pltpu.VMEM((_CCH, D), F32),
            pltpu.VMEM((_CCH, D), F32),
            pltpu.VMEM((_CCH, D), F32),
            pltpu.VMEM((_TPW,), jnp.int32),
            pltpu.VMEM((_TPW,), jnp.int32),
            pltpu.VMEM((_TPW, _L), F32),
            pltpu.VMEM((_TPW, _L), F32),
            pltpu.VMEM((_CCH,), jnp.int32),
            pltpu.VMEM((_CCH,), jnp.int32),
            pltpu.SemaphoreType.DMA,
            pltpu.SemaphoreType.DMA,
            pltpu.SemaphoreType.DMA,
        ],
    )
    def k(sh_hbm, y_hbm, p0_hbm, p1_hbm, wae_hbm, wbe_hbm, out_hbm,
          sh_v, y0_v, y1_v, i0all_v, i1all_v, wa_v, wb_v, i0_v, i1_v,
          sem0, sem1, osem):
        wid = lax.axis_index("s") * _NC + lax.axis_index("c")
        base = wid * _TPW
        pltpu.sync_copy(p0_hbm.at[pl.ds(base, _TPW)], i0all_v)
        pltpu.sync_copy(p1_hbm.at[pl.ds(base, _TPW)], i1all_v)
        pltpu.sync_copy(wae_hbm.at[pl.ds(base, _TPW)], wa_v)
        pltpu.sync_copy(wbe_hbm.at[pl.ds(base, _TPW)], wb_v)
        ocp = None
        for c in range(_TPW // _CCH):
            t0 = base + c * _CCH
            i0_v[...] = i0all_v[pl.ds(c * _CCH, _CCH)]
            i1_v[...] = i1all_v[pl.ds(c * _CCH, _CCH)]
            cp0 = pltpu.async_copy(y_hbm.at[i0_v], y0_v, sem0)
            cp1 = pltpu.async_copy(y_hbm.at[i1_v], y1_v, sem1)
            if ocp is not None:
                ocp.wait()
            pltpu.sync_copy(sh_hbm.at[pl.ds(t0, _CCH)], sh_v)
            cp0.wait()
            cp1.wait()

            def row(r, _2):
                g = c * _CCH + r
                w0 = wa_v[g, :]
                w1v = wb_v[g, :]
                for kk in range(D // _L):
                    sl = pl.ds(kk * _L, _L)
                    sh_v[r, sl] = (sh_v[r, sl] + w0 * y0_v[r, sl]
                                   + w1v * y1_v[r, sl])
                return 0

            lax.fori_loop(0, _CCH, row, 0)
            ocp = pltpu.async_copy(sh_v, out_hbm.at[pl.ds(t0, _CCH)], osem)
        ocp.wait()

    return k(shared, y, p0, p1, wae, wbe)


def _dispatch_plan(e1, e2, wA, wB):
    ef = jnp.stack([e1, e2], axis=1).reshape(-1)
    oh = (ef[:, None] == jnp.arange(E)[None, :]).astype(jnp.int32)
    ranks = jnp.cumsum(oh, axis=0)
    rank = jnp.sum(ranks * oh, axis=1) - 1
    counts = ranks[-1]
    nblk = (counts + T - 1) // T
    ends = jnp.cumsum(nblk)
    startblk = ends - nblk
    ppos = (startblk[ef] * T + rank).astype(jnp.int32)
    p0 = ppos[0::2]
    p1 = ppos[1::2]
    tvec = jnp.arange(MAXB, dtype=jnp.int32)
    used = ends[-1]
    te = jnp.minimum(jnp.searchsorted(ends, tvec, side="right"),
                     E - 1).astype(jnp.int32)
    tv = (tvec < used).astype(jnp.int32)
    ob = jnp.where(tv == 1, tvec, MAXB).astype(jnp.int32)
    return p0, p1, te, tv, ob


def kernel(hidden_states, wg, shared_gate_w, shared_up_w, shared_down_w, w1, w2, w3):
    B = hidden_states.shape[0]
    x = hidden_states.reshape(S, D)

    xb, e1, e2, wfa, wfb = _router(x, wg)
    p0, p1, te, tv, ob = _dispatch_plan(e1, e2, wfa, wfb)

    xg = _sc_dispatch(x, p0, p1)

    hs = _shared_h(xb, shared_gate_w, shared_up_w)
    shared = _shared_out(hs, shared_down_w)

    h = _grouped_h(xg, w1, w3, te, ob, tv)
    y = _grouped_out(h, w2, te, ob, tv)

    wae = jnp.broadcast_to(wfa[:, None], (S, _L))
    wbe = jnp.broadcast_to(wfb[:, None], (S, _L))
    return _sc_combine(shared, y, p0, p1, wae, wbe).reshape(B, S, D)
